# baseline (speedup 1.0000x reference)
import jax
import jax.numpy as jnp
from jax import lax
from jax.experimental import pallas as pl
from jax.experimental.pallas import tpu as pltpu
from jax.experimental.pallas import tpu_sc as plsc

S, B, T, D = 20, 1024, 26, 128
N = S * B * T
NW = 32
PER_W = N // NW
SLABS = 4
NR = SLABS * T
NCH = PER_W // NR
SLABS_W = PER_W // T


def _body(tags_hbm, probs_hbm, table_hbm, out3_hbm,
          idx_all, prob_all, rows0, rows1, ob0, ob1,
          gsem0, gsem1, osem0, osem1):
    c = lax.axis_index("c")
    s = lax.axis_index("s")
    wid = s * 2 + c
    base = wid * PER_W
    slab0 = wid * SLABS_W

    rows = (rows0, rows1)
    obuf = (ob0, ob1)
    gsem = (gsem0, gsem1)
    osem = (osem0, osem1)

    pltpu.sync_copy(tags_hbm.at[pl.ds(base, PER_W)], idx_all)
    pltpu.sync_copy(probs_hbm.at[pl.ds(base, PER_W)], prob_all)

    def fire_gather(b, g):
        pltpu.async_copy(table_hbm.at[idx_all.at[pl.ds(g * NR, NR)]],
                         rows[b], gsem[b])

    def wait_gather(b):
        pltpu.make_async_copy(table_hbm.at[pl.ds(0, NR)], rows[b], gsem[b]).wait()

    def fire_out(b, g):
        pltpu.async_copy(obuf[b], out3_hbm.at[pl.ds(slab0 + g * SLABS, SLABS)],
                         osem[b])

    def wait_out(b):
        pltpu.make_async_copy(obuf[b], out3_hbm.at[pl.ds(0, SLABS)],
                              osem[b]).wait()

    def compute(b, g):
        def slab(i, carry):
            off = g * NR + i * T
            pv0 = prob_all[pl.ds(off, 16)]
            pv1 = prob_all[pl.ds(off + T - 16, 16)]
            for t in range(T):
                pb = pv0[t] if t < 16 else pv1[t - (T - 16)]
                r = i * T + t
                for j in range(8):
                    sl = pl.ds(j * 16, 16)
                    obuf[b][i, t, sl] = rows[b][r, sl] * pb
            return carry
        lax.fori_loop(0, SLABS, slab, 0)

    fire_gather(0, 0)
    fire_gather(1, 1)

    for b in range(2):
        wait_gather(b)
        compute(b, b)
        fire_out(b, b)
        fire_gather(b, b + 2)

    def pair(qq, carry):
        for b in range(2):
            g = qq * 2 + b
            wait_gather(b)
            wait_out(b)
            compute(b, g)
            fire_out(b, g)
            gn = jnp.minimum(g + 2, NCH - 1)
            fire_gather(b, gn)
        return carry

    lax.fori_loop(1, NCH // 2, pair, 0)

    for b in range(2):
        wait_gather(b)
        wait_out(b)


@jax.jit
def _run(tags_flat, probs_flat, table):
    mesh = plsc.VectorSubcoreMesh(core_axis_name="c", subcore_axis_name="s")
    out = pl.kernel(
        _body,
        out_type=jax.ShapeDtypeStruct((S * B, T, D), jnp.float32),
        mesh=mesh,
        scratch_types=[
            pltpu.VMEM((PER_W,), jnp.int32),
            pltpu.VMEM((PER_W,), jnp.float32),
            pltpu.VMEM((NR, D), jnp.float32),
            pltpu.VMEM((NR, D), jnp.float32),
            pltpu.VMEM((SLABS, T, D), jnp.float32),
            pltpu.VMEM((SLABS, T, D), jnp.float32),
            pltpu.SemaphoreType.DMA,
            pltpu.SemaphoreType.DMA,
            pltpu.SemaphoreType.DMA,
            pltpu.SemaphoreType.DMA,
        ],
    )(tags_flat, probs_flat, table)
    return out


def kernel(tags, probs, table):
    out = _run(tags.reshape(N), probs.reshape(N), table)
    return out.reshape(S, B, T, D)


# speedup vs baseline: 1.0015x; 1.0015x over previous
import jax
import jax.numpy as jnp
from jax import lax
from jax.experimental import pallas as pl
from jax.experimental.pallas import tpu as pltpu
from jax.experimental.pallas import tpu_sc as plsc

S, B, T, D = 20, 1024, 26, 128
N = S * B * T
NW = 32
PER_W = N // NW
SLABS = 4
NR = SLABS * T
NCH = PER_W // NR
SLABS_W = PER_W // T


def _body(tags_hbm, probs_hbm, table_hbm, out3_hbm,
          idx_all, prob_all, rows0, rows1, ob0, ob1,
          gsem0, gsem1, osem0, osem1):
    c = lax.axis_index("c")
    s = lax.axis_index("s")
    wid = s * 2 + c
    base = wid * PER_W
    slab0 = wid * SLABS_W

    rows = (rows0, rows1)
    obuf = (ob0, ob1)
    gsem = (gsem0, gsem1)
    osem = (osem0, osem1)

    pltpu.sync_copy(tags_hbm.at[pl.ds(base, PER_W)], idx_all)
    pltpu.sync_copy(probs_hbm.at[pl.ds(base, PER_W)], prob_all)

    def fire_gather(b, g):
        pltpu.async_copy(table_hbm.at[idx_all.at[pl.ds(g * NR, NR)]],
                         rows[b], gsem[b])

    def wait_gather(b):
        pltpu.make_async_copy(table_hbm.at[pl.ds(0, NR)], rows[b],
                              gsem[b]).wait()

    def fire_out(b, g):
        pltpu.async_copy(obuf[b], out3_hbm.at[pl.ds(slab0 + g * SLABS, SLABS)],
                         osem[b])

    def wait_out(b):
        pltpu.make_async_copy(obuf[b], out3_hbm.at[pl.ds(0, SLABS)],
                              osem[b]).wait()

    def compute(b, g):
        def slab(i, carry):
            off = g * NR + i * T
            pv0 = prob_all[pl.ds(off, 16)]
            pv1 = prob_all[pl.ds(off + T - 16, 16)]
            for t in range(T):
                pb = pv0[t] if t < 16 else pv1[t - (T - 16)]
                r = i * T + t
                for j in range(8):
                    sl = pl.ds(j * 16, 16)
                    obuf[b][i, t, sl] = rows[b][r, sl] * pb
            return carry
        lax.fori_loop(0, SLABS, slab, 0)

    fire_gather(0, 0)
    fire_gather(1, 1)

    for b in range(2):
        wait_gather(b)
        compute(b, b)
        fire_out(b, b)
        fire_gather(b, b + 2)

    def pair(qq, carry):
        for b in range(2):
            g = qq * 2 + b
            wait_gather(b)
            wait_out(b)
            compute(b, g)
            fire_out(b, g)
            gn = jnp.minimum(g + 2, NCH - 1)
            fire_gather(b, gn)
        return carry

    lax.fori_loop(1, NCH // 2, pair, 0)

    for b in range(2):
        wait_gather(b)
        wait_out(b)


@jax.jit
def _run(tags_flat, probs_flat, table):
    mesh = plsc.VectorSubcoreMesh(core_axis_name="c", subcore_axis_name="s")
    out = pl.kernel(
        _body,
        out_type=jax.ShapeDtypeStruct((S * B, T, D), jnp.float32),
        mesh=mesh,
        compiler_params=pltpu.CompilerParams(use_tc_tiling_on_sc=True),
        scratch_types=[
            pltpu.VMEM((PER_W,), jnp.int32),
            pltpu.VMEM((PER_W,), jnp.float32),
            pltpu.VMEM((NR, D), jnp.float32),
            pltpu.VMEM((NR, D), jnp.float32),
            pltpu.VMEM((SLABS, T, D), jnp.float32),
            pltpu.VMEM((SLABS, T, D), jnp.float32),
            pltpu.SemaphoreType.DMA,
            pltpu.SemaphoreType.DMA,
            pltpu.SemaphoreType.DMA,
            pltpu.SemaphoreType.DMA,
        ],
    )(tags_flat, probs_flat, table)
    return out


def kernel(tags, probs, table):
    out = _run(tags.reshape(N), probs.reshape(N), table)
    return out.reshape(S, B, T, D)


# speedup vs baseline: 4.7699x; 4.7627x over previous
import jax
import jax.numpy as jnp
from jax import lax
from jax.experimental import pallas as pl
from jax.experimental.pallas import tpu as pltpu
from jax.experimental.pallas import tpu_sc as plsc

S, B, T, D = 20, 1024, 26, 128
N = S * B * T
NW = 32
PER_W = N // NW
CH = 128
NCH = PER_W // CH


def _body(tags_hbm, probs_hbm, table_hbm, out_hbm,
          idx_all, prob_all, rows0, rows1, ob0, ob1,
          gsem0, gsem1, osem0, osem1):
    c = lax.axis_index("c")
    s = lax.axis_index("s")
    wid = s * 2 + c
    base = wid * PER_W

    rows = (rows0, rows1)
    obuf = (ob0, ob1)
    gsem = (gsem0, gsem1)
    osem = (osem0, osem1)

    pltpu.sync_copy(tags_hbm.at[pl.ds(base, PER_W)], idx_all)
    pltpu.sync_copy(probs_hbm.at[pl.ds(base, PER_W)], prob_all)

    def fire_gather(b, g):
        pltpu.async_copy(table_hbm.at[idx_all.at[pl.ds(g * CH, CH)]],
                         rows[b], gsem[b])

    def wait_gather(b):
        pltpu.make_async_copy(table_hbm.at[pl.ds(0, CH)], rows[b],
                              gsem[b]).wait()

    def fire_out(b, g):
        pltpu.async_copy(obuf[b], out_hbm.at[pl.ds(base + g * CH, CH)], osem[b])

    def wait_out(b):
        pltpu.make_async_copy(obuf[b], out_hbm.at[pl.ds(0, CH)], osem[b]).wait()

    def compute(b, g):
        def group(q, carry):
            pv = prob_all[pl.ds(g * CH + q * 16, 16)]
            for k in range(16):
                r = q * 16 + k
                pb = pv[k]
                for j in range(8):
                    sl = pl.ds(j * 16, 16)
                    obuf[b][r, sl] = rows[b][r, sl] * pb
            return carry
        lax.fori_loop(0, CH // 16, group, 0)

    fire_gather(0, 0)
    fire_gather(1, 1)

    for b in range(2):
        wait_gather(b)
        compute(b, b)
        fire_out(b, b)
        fire_gather(b, b + 2)

    def pair(qq, carry):
        for b in range(2):
            g = qq * 2 + b
            wait_gather(b)
            wait_out(b)
            compute(b, g)
            fire_out(b, g)
            gn = jnp.minimum(g + 2, NCH - 1)
            fire_gather(b, gn)
        return carry

    lax.fori_loop(1, NCH // 2, pair, 0)

    for b in range(2):
        wait_gather(b)
        wait_out(b)


@jax.jit
def _run(tags_flat, probs_flat, table):
    mesh = plsc.VectorSubcoreMesh(core_axis_name="c", subcore_axis_name="s")
    out = pl.kernel(
        _body,
        out_type=jax.ShapeDtypeStruct((N, D), jnp.float32),
        mesh=mesh,
        scratch_types=[
            pltpu.VMEM((PER_W,), jnp.int32),
            pltpu.VMEM((PER_W,), jnp.float32),
            pltpu.VMEM((CH, D), jnp.float32),
            pltpu.VMEM((CH, D), jnp.float32),
            pltpu.VMEM((CH, D), jnp.float32),
            pltpu.VMEM((CH, D), jnp.float32),
            pltpu.SemaphoreType.DMA,
            pltpu.SemaphoreType.DMA,
            pltpu.SemaphoreType.DMA,
            pltpu.SemaphoreType.DMA,
        ],
    )(tags_flat, probs_flat, table)
    return out


def kernel(tags, probs, table):
    tt = tags.transpose(0, 2, 1).reshape(N)
    pp = probs.transpose(0, 2, 1).reshape(N)
    out = _run(tt, pp, table)
    return out.reshape(S, T, B, D).transpose(0, 2, 1, 3)


# speedup vs baseline: 4.9043x; 1.0282x over previous
import jax
import jax.numpy as jnp
from jax import lax
from jax.experimental import pallas as pl
from jax.experimental.pallas import tpu as pltpu
from jax.experimental.pallas import tpu_sc as plsc

S, B, T, D = 20, 1024, 26, 128
N = S * B * T
NW = 32
PER_W = N // NW
CH = 128
NCH = PER_W // CH


def _body(tags_hbm, probs_hbm, table_hbm, out_hbm,
          idx_all, prob_all, rows0, rows1, ob0, ob1,
          gsem0, gsem1, osem0, osem1):
    c = lax.axis_index("c")
    s = lax.axis_index("s")
    wid = s * 2 + c
    base = wid * PER_W

    rows = (rows0, rows1)
    obuf = (ob0, ob1)
    gsem = (gsem0, gsem1)
    osem = (osem0, osem1)

    pltpu.sync_copy(tags_hbm.at[pl.ds(base, PER_W)], idx_all)
    pltpu.sync_copy(probs_hbm.at[pl.ds(base, PER_W)], prob_all)

    def fire_gather(b, g):
        pltpu.async_copy(table_hbm.at[idx_all.at[pl.ds(g * CH, CH)]],
                         rows[b], gsem[b])

    def wait_gather(b):
        pltpu.make_async_copy(table_hbm.at[pl.ds(0, CH)], rows[b],
                              gsem[b]).wait()

    def fire_out(b, g):
        pltpu.async_copy(obuf[b], out_hbm.at[pl.ds(base + g * CH, CH)], osem[b])

    def wait_out(b):
        pltpu.make_async_copy(obuf[b], out_hbm.at[pl.ds(0, CH)], osem[b]).wait()

    def compute(b, g):
        def group(q, carry):
            pv = prob_all[pl.ds(g * CH + q * 16, 16)]
            for k in range(16):
                r = q * 16 + k
                pb = pv[k]
                for j in range(8):
                    sl = pl.ds(j * 16, 16)
                    obuf[b][r, sl] = rows[b][r, sl] * pb
            return carry
        lax.fori_loop(0, CH // 16, group, 0, unroll=2)

    fire_gather(0, 0)
    fire_gather(1, 1)

    for b in range(2):
        wait_gather(b)
        compute(b, b)
        fire_out(b, b)
        fire_gather(b, b + 2)

    def pair(qq, carry):
        for b in range(2):
            g = qq * 2 + b
            wait_gather(b)
            wait_out(b)
            compute(b, g)
            fire_out(b, g)
            gn = jnp.minimum(g + 2, NCH - 1)
            fire_gather(b, gn)
        return carry

    lax.fori_loop(1, NCH // 2, pair, 0)

    for b in range(2):
        wait_gather(b)
        wait_out(b)


@jax.jit
def _run(tags_flat, probs_flat, table):
    mesh = plsc.VectorSubcoreMesh(core_axis_name="c", subcore_axis_name="s")
    out = pl.kernel(
        _body,
        out_type=jax.ShapeDtypeStruct((N, D), jnp.float32),
        mesh=mesh,
        scratch_types=[
            pltpu.VMEM((PER_W,), jnp.int32),
            pltpu.VMEM((PER_W,), jnp.float32),
            pltpu.VMEM((CH, D), jnp.float32),
            pltpu.VMEM((CH, D), jnp.float32),
            pltpu.VMEM((CH, D), jnp.float32),
            pltpu.VMEM((CH, D), jnp.float32),
            pltpu.SemaphoreType.DMA,
            pltpu.SemaphoreType.DMA,
            pltpu.SemaphoreType.DMA,
            pltpu.SemaphoreType.DMA,
        ],
    )(tags_flat, probs_flat, table)
    return out


def kernel(tags, probs, table):
    tt = tags.transpose(0, 2, 1).reshape(N)
    pp = probs.transpose(0, 2, 1).reshape(N)
    out = _run(tt, pp, table)
    return out.reshape(S, T, B, D).transpose(0, 2, 1, 3)
